# baseline (speedup 1.0000x reference)
import functools

import jax
import jax.numpy as jnp
from jax import lax
from jax.experimental import pallas as pl
from jax.experimental.pallas import tpu as pltpu
from jax.experimental.pallas import tpu_sc as plsc

_D = 64
_B = 4096 * 200
_NW = 32
_BPW = _B // _NW
_CHUNK = 512
_STREAM = 128
_NSTREAM = _CHUNK // _STREAM
_NCHUNKS = _BPW // _CHUNK


def _body(table_hbm, x_hbm, out_hbm, idx_v, rows_v, gsem0, gsem1):
    wid = lax.axis_index("s") * 2 + lax.axis_index("c")
    base = wid * _BPW
    pltpu.sync_copy(x_hbm.at[pl.ds(base, _BPW)], idx_v)

    gsems = (gsem0, gsem1)

    def start_gather(g, slot):
        off = pl.multiple_of(g * _CHUNK, _CHUNK)
        for j in range(_NSTREAM):
            pltpu.async_copy(
                table_hbm.at[idx_v.at[pl.ds(off + j * _STREAM, _STREAM)]],
                rows_v.at[slot, pl.ds(j * _STREAM, _STREAM)],
                gsems[slot],
            )

    def wait_gather(slot):
        for j in range(_NSTREAM):
            pltpu.make_async_copy(
                table_hbm.at[pl.ds(0, _STREAM)],
                rows_v.at[slot, pl.ds(j * _STREAM, _STREAM)],
                gsems[slot],
            ).wait()

    def write_out(g, slot):
        off = pl.multiple_of(g * _CHUNK, _CHUNK)
        pltpu.sync_copy(rows_v.at[slot], out_hbm.at[pl.ds(base + off, _CHUNK)])

    start_gather(0, 0)

    @pl.loop(0, _NCHUNKS - 2, step=2)
    def _(g):
        start_gather(g + 1, 1)
        wait_gather(0)
        write_out(g, 0)
        start_gather(g + 2, 0)
        wait_gather(1)
        write_out(g + 1, 1)

    start_gather(_NCHUNKS - 1, 1)
    wait_gather(0)
    write_out(_NCHUNKS - 2, 0)
    wait_gather(1)
    write_out(_NCHUNKS - 1, 1)


_embed_gather = functools.partial(
    pl.kernel,
    out_type=jax.ShapeDtypeStruct((_B, _D), jnp.float32),
    mesh=plsc.VectorSubcoreMesh(core_axis_name="c", subcore_axis_name="s"),
    scratch_types=[
        pltpu.VMEM((_BPW,), jnp.int32),
        pltpu.VMEM((2, _CHUNK, _D), jnp.float32),
        pltpu.SemaphoreType.DMA,
        pltpu.SemaphoreType.DMA,
    ],
    compiler_params=pltpu.CompilerParams(use_tc_tiling_on_sc=False),
)(_body)


def kernel(x, table):
    flat = _embed_gather(table, x.reshape(-1))
    return flat.reshape(x.shape + (table.shape[1],))


# speedup vs baseline: 1.2221x; 1.2221x over previous
import functools

import jax
import jax.numpy as jnp
from jax import lax
from jax.experimental import pallas as pl
from jax.experimental.pallas import tpu as pltpu
from jax.experimental.pallas import tpu_sc as plsc

_D = 128
_B = 4096 * 200
_NW = 32
_BPW = _B // _NW
_CHUNK = 256
_STREAM = 128
_NSTREAM = _CHUNK // _STREAM
_NCHUNKS = _BPW // _CHUNK


def _body(table_hbm, x_hbm, out_hbm, idx_v, rows_v, gsem0, gsem1):
    wid = lax.axis_index("s") * 2 + lax.axis_index("c")
    base = wid * _BPW
    pltpu.sync_copy(x_hbm.at[pl.ds(base, _BPW)], idx_v)

    gsems = (gsem0, gsem1)

    def start_gather(g, slot):
        off = pl.multiple_of(g * _CHUNK, _CHUNK)
        for j in range(_NSTREAM):
            pltpu.async_copy(
                table_hbm.at[idx_v.at[pl.ds(off + j * _STREAM, _STREAM)]],
                rows_v.at[slot, pl.ds(j * _STREAM, _STREAM)],
                gsems[slot],
            )

    def wait_gather(slot):
        for j in range(_NSTREAM):
            pltpu.make_async_copy(
                table_hbm.at[pl.ds(0, _STREAM)],
                rows_v.at[slot, pl.ds(j * _STREAM, _STREAM)],
                gsems[slot],
            ).wait()

    def write_out(g, slot):
        off = pl.multiple_of(g * _CHUNK, _CHUNK)
        pltpu.sync_copy(rows_v.at[slot], out_hbm.at[pl.ds(base + off, _CHUNK)])

    start_gather(0, 0)

    @pl.loop(0, _NCHUNKS - 2, step=2)
    def _(g):
        start_gather(g + 1, 1)
        wait_gather(0)
        write_out(g, 0)
        start_gather(g + 2, 0)
        wait_gather(1)
        write_out(g + 1, 1)

    start_gather(_NCHUNKS - 1, 1)
    wait_gather(0)
    write_out(_NCHUNKS - 2, 0)
    wait_gather(1)
    write_out(_NCHUNKS - 1, 1)


_embed_gather = functools.partial(
    pl.kernel,
    out_type=jax.ShapeDtypeStruct((_B, _D), jnp.float32),
    mesh=plsc.VectorSubcoreMesh(core_axis_name="c", subcore_axis_name="s"),
    scratch_types=[
        pltpu.VMEM((_BPW,), jnp.int32),
        pltpu.VMEM((2, _CHUNK, _D), jnp.float32),
        pltpu.SemaphoreType.DMA,
        pltpu.SemaphoreType.DMA,
    ],
    compiler_params=pltpu.CompilerParams(use_tc_tiling_on_sc=False),
)(_body)


def kernel(x, table):
    table128 = jnp.pad(table, ((0, 0), (0, _D - table.shape[1])))
    flat = _embed_gather(table128, x.reshape(-1))
    return flat[:, : table.shape[1]].reshape(x.shape + (table.shape[1],))


# speedup vs baseline: 1.3184x; 1.0787x over previous
import functools

import jax
import jax.numpy as jnp
from jax import lax
from jax.experimental import pallas as pl
from jax.experimental.pallas import tpu as pltpu
from jax.experimental.pallas import tpu_sc as plsc

_D = 64
_DP = 128
_B = 4096 * 200
_NW = 32
_BPW = _B // _NW
_CHUNK = 256
_STREAM = 128
_NSTREAM = _CHUNK // _STREAM
_NCHUNKS = _BPW // _CHUNK


def _body(table_hbm, x_hbm, out_hbm, idx_v, rows_v, gsem0, gsem1):
    wid = lax.axis_index("s") * 2 + lax.axis_index("c")
    base = wid * _BPW
    pltpu.sync_copy(x_hbm.at[pl.ds(base, _BPW)], idx_v)

    gsems = (gsem0, gsem1)

    def start_gather(g, slot):
        off = pl.multiple_of(g * _CHUNK, _CHUNK)
        for j in range(_NSTREAM):
            pltpu.async_copy(
                table_hbm.at[idx_v.at[pl.ds(off + j * _STREAM, _STREAM)]],
                rows_v.at[slot, pl.ds(j * _STREAM, _STREAM)],
                gsems[slot],
            )

    def wait_gather(slot):
        for j in range(_NSTREAM):
            pltpu.make_async_copy(
                table_hbm.at[pl.ds(0, _STREAM)],
                rows_v.at[slot, pl.ds(j * _STREAM, _STREAM)],
                gsems[slot],
            ).wait()

    def write_out(g, slot):
        off = pl.multiple_of(g * _CHUNK, _CHUNK)
        pltpu.sync_copy(
            rows_v.at[slot, slice(None), pl.ds(0, _D)],
            out_hbm.at[pl.ds(base + off, _CHUNK), pl.ds(0, _D)],
        )

    start_gather(0, 0)

    @pl.loop(0, _NCHUNKS - 2, step=2)
    def _(g):
        start_gather(g + 1, 1)
        wait_gather(0)
        write_out(g, 0)
        start_gather(g + 2, 0)
        wait_gather(1)
        write_out(g + 1, 1)

    start_gather(_NCHUNKS - 1, 1)
    wait_gather(0)
    write_out(_NCHUNKS - 2, 0)
    wait_gather(1)
    write_out(_NCHUNKS - 1, 1)


_embed_gather = functools.partial(
    pl.kernel,
    out_type=jax.ShapeDtypeStruct((_B, _DP), jnp.float32),
    mesh=plsc.VectorSubcoreMesh(core_axis_name="c", subcore_axis_name="s"),
    scratch_types=[
        pltpu.VMEM((_BPW,), jnp.int32),
        pltpu.VMEM((2, _CHUNK, _DP), jnp.float32),
        pltpu.SemaphoreType.DMA,
        pltpu.SemaphoreType.DMA,
    ],
    compiler_params=pltpu.CompilerParams(use_tc_tiling_on_sc=False),
)(_body)


def kernel(x, table):
    table128 = jnp.pad(table, ((0, 0), (0, _DP - table.shape[1])))
    flat = _embed_gather(table128, x.reshape(-1))
    return flat[:, : table.shape[1]].reshape(x.shape + (table.shape[1],))


# speedup vs baseline: 1.3310x; 1.0096x over previous
import functools

import jax
import jax.numpy as jnp
from jax import lax
from jax.experimental import pallas as pl
from jax.experimental.pallas import tpu as pltpu
from jax.experimental.pallas import tpu_sc as plsc

_D = 64
_DP = 128
_B = 4096 * 200
_NW = 32
_BPW = _B // _NW
_CHUNK = 512
_STREAM = 128
_NSTREAM = _CHUNK // _STREAM
_NCHUNKS = _BPW // _CHUNK


def _body(table_hbm, x_hbm, out_hbm, idx_v, rows_v, gsem0, gsem1):
    wid = lax.axis_index("s") * 2 + lax.axis_index("c")
    base = wid * _BPW
    pltpu.sync_copy(x_hbm.at[pl.ds(base, _BPW)], idx_v)

    gsems = (gsem0, gsem1)

    def start_gather(g, slot):
        off = pl.multiple_of(g * _CHUNK, _CHUNK)
        for j in range(_NSTREAM):
            pltpu.async_copy(
                table_hbm.at[idx_v.at[pl.ds(off + j * _STREAM, _STREAM)]],
                rows_v.at[slot, pl.ds(j * _STREAM, _STREAM)],
                gsems[slot],
            )

    def wait_gather(slot):
        for j in range(_NSTREAM):
            pltpu.make_async_copy(
                table_hbm.at[pl.ds(0, _STREAM)],
                rows_v.at[slot, pl.ds(j * _STREAM, _STREAM)],
                gsems[slot],
            ).wait()

    def write_out(g, slot):
        off = pl.multiple_of(g * _CHUNK, _CHUNK)
        pltpu.sync_copy(
            rows_v.at[slot],
            out_hbm.at[pl.ds(base + off, _CHUNK), pl.ds(0, _D)],
        )

    start_gather(0, 0)

    @pl.loop(0, _NCHUNKS - 2, step=2)
    def _(g):
        start_gather(g + 1, 1)
        wait_gather(0)
        write_out(g, 0)
        start_gather(g + 2, 0)
        wait_gather(1)
        write_out(g + 1, 1)

    start_gather(_NCHUNKS - 1, 1)
    wait_gather(0)
    write_out(_NCHUNKS - 2, 0)
    wait_gather(1)
    write_out(_NCHUNKS - 1, 1)


_embed_gather = functools.partial(
    pl.kernel,
    out_type=jax.ShapeDtypeStruct((_B, _DP), jnp.float32),
    mesh=plsc.VectorSubcoreMesh(core_axis_name="c", subcore_axis_name="s"),
    scratch_types=[
        pltpu.VMEM((_BPW,), jnp.int32),
        pltpu.VMEM((2, _CHUNK, _D), jnp.float32),
        pltpu.SemaphoreType.DMA,
        pltpu.SemaphoreType.DMA,
    ],
    compiler_params=pltpu.CompilerParams(use_tc_tiling_on_sc=False),
)(_body)


def kernel(x, table):
    flat = _embed_gather(table, x.reshape(-1))
    return flat[:, : table.shape[1]].reshape(x.shape + (table.shape[1],))
